# Initial kernel scaffold; baseline (speedup 1.0000x reference)
import jax
import jax.numpy as jnp
from jax.experimental import pallas as pl


def kernel(batch_indices, features, neighborhood_ids, candidates, unit_neighb_lut, Coo_logdet, Coo_inv, Coinv_Com, obs_ix, miss_ix, nobs, log_proportions, nu, tnu, Wobs, Cooinv_nu, obs_logdets, Cobsinv_WobsT, T, W_WCC, inv_cap, noise_logliks):
    raise NotImplementedError("write your pallas kernel here")


# speedup vs baseline: 1.2964x; 1.2964x over previous
import functools
import math

import jax
import jax.numpy as jnp
from jax.experimental import pallas as pl
from jax.experimental.pallas import tpu as pltpu

B = 2048
DO = 96
DM = 96
C = 4
M = 8

BLK = 64
LOG2PI = math.log(2.0 * math.pi)


def _te_block(inv_ref, xo_ref, xm_ref, ci_ref, cicm_ref, nu_ref, tnu_ref,
              cinu_ref, wo_ref, ciwt_ref, ww_ref, tb_ref, aux_ref, ldets_ref,
              lp_ref, out_ref):
    inv_cap = inv_ref[0]
    xo = xo_ref[:]
    xm = xm_ref[:]
    ci = ci_ref[:]
    cicm = cicm_ref[:]
    ld = aux_ref[:, 0]
    nob = aux_ref[:, 1]
    nll = aux_ref[:, 2]

    xCx = jnp.sum(ci * xo[:, :, None] * xo[:, None, :], axis=(1, 2))
    base = -0.5 * (ld + nob * LOG2PI) - nll

    for c in range(C):
        nu_c = nu_ref[:, c, :]
        cinu_c = cinu_ref[:, c, :]
        dx = xo - nu_c
        mahal = xCx - 2.0 * jnp.sum(xo * cinu_c, axis=1) \
            + jnp.sum(nu_c * cinu_c, axis=1)
        p = jnp.sum(ciwt_ref[:, c, :, :] * dx[:, None, :], axis=2)
        corr = jnp.sum(p[:, :, None] * tb_ref[:, c, :, :] * p[:, None, :],
                       axis=(1, 2))
        em = tnu_ref[:, c, :] + jnp.sum(cicm * dx[:, :, None], axis=1)
        r = xm - em
        wq = jnp.sum(ww_ref[:, c, :, :] * dx[:, None, :], axis=2)
        woq = jnp.sum(wo_ref[:, c, :, :] * dx[:, None, :], axis=2)
        lls = base - 0.5 * (ldets_ref[:, c] + mahal - corr) + lp_ref[:, c]
        lls = lls - 0.5 * inv_cap * jnp.sum(r * r, axis=1)
        lls = lls + jnp.sum(wq * p, axis=1)
        lls = lls + 0.01 * jnp.sum(woq * p, axis=1)
        out_ref[:, c] = lls


def kernel(batch_indices, features, neighborhood_ids, candidates,
           unit_neighb_lut, Coo_logdet, Coo_inv, Coinv_Com, obs_ix, miss_ix,
           nobs, log_proportions, nu, tnu, Wobs, Cooinv_nu, obs_logdets,
           Cobsinv_WobsT, T, W_WCC, inv_cap, noise_logliks):
    x = features[batch_indices]
    nb = neighborhood_ids[batch_indices]
    cand = candidates[batch_indices]
    lut = unit_neighb_lut[cand, nb[:, None]]
    xo = jnp.take_along_axis(x, obs_ix[nb], axis=1)
    xm = jnp.take_along_axis(x, miss_ix[nb], axis=1)

    ci = Coo_inv[nb]
    cicm = Coinv_Com[nb]
    nu_b = nu[lut]
    tnu_b = tnu[lut]
    cinu_b = Cooinv_nu[lut]
    wo_b = jnp.swapaxes(Wobs[lut], 2, 3)
    ciwt_b = Cobsinv_WobsT[lut]
    ww_b = jnp.swapaxes(W_WCC[lut], 2, 3)
    tb_b = T[lut]

    ld = Coo_logdet[nb]
    ldets = obs_logdets[lut]
    lp = log_proportions[cand]
    nob = nobs[nb].astype(jnp.float32)
    nll = noise_logliks[batch_indices]
    aux = jnp.stack([ld, nob, nll, jnp.zeros_like(ld)], axis=1)
    inv_arr = jnp.reshape(inv_cap, (1,)).astype(jnp.float32)

    nblk = B // BLK
    spec = lambda bs, im: pl.BlockSpec(bs, im)
    grid_spec = pl.GridSpec(
        grid=(nblk,),
        in_specs=[
            pl.BlockSpec(memory_space=pltpu.SMEM),
            spec((BLK, DO), lambda i: (i, 0)),
            spec((BLK, DM), lambda i: (i, 0)),
            spec((BLK, DO, DO), lambda i: (i, 0, 0)),
            spec((BLK, DO, DM), lambda i: (i, 0, 0)),
            spec((BLK, C, DO), lambda i: (i, 0, 0)),
            spec((BLK, C, DM), lambda i: (i, 0, 0)),
            spec((BLK, C, DO), lambda i: (i, 0, 0)),
            spec((BLK, C, M, DO), lambda i: (i, 0, 0, 0)),
            spec((BLK, C, M, DO), lambda i: (i, 0, 0, 0)),
            spec((BLK, C, M, DO), lambda i: (i, 0, 0, 0)),
            spec((BLK, C, M, M), lambda i: (i, 0, 0, 0)),
            spec((BLK, 4), lambda i: (i, 0)),
            spec((BLK, C), lambda i: (i, 0)),
            spec((BLK, C), lambda i: (i, 0)),
        ],
        out_specs=spec((BLK, C), lambda i: (i, 0)),
    )
    return pl.pallas_call(
        _te_block,
        grid_spec=grid_spec,
        out_shape=jax.ShapeDtypeStruct((B, C), jnp.float32),
    )(inv_arr, xo, xm, ci, cicm, nu_b, tnu_b, cinu_b, wo_b, ciwt_b, ww_b,
      tb_b, aux, ldets, lp)
